# baseline (speedup 1.0000x reference)
import jax
import jax.numpy as jnp
from jax import lax
from jax.experimental import pallas as pl
from jax.experimental.pallas import tpu as pltpu
from jax.experimental.pallas import tpu_sc as plsc

_DIM = 64
_PAD = 128
_SCALE = 8.0
_L = 16
_NW = 32
_GCHUNK = 128
_NBUF = 2


def _transpose_scale_pad(tab_t, tail_pad):
    d, v = tab_t.shape
    ngfull = v // _PAD
    tail = v - ngfull * _PAD
    mesh = plsc.VectorSubcoreMesh(core_axis_name="c", subcore_axis_name="s")

    @pl.kernel(
        out_type=jax.ShapeDtypeStruct((v, _PAD), jnp.float32),
        mesh=mesh,
        compiler_params=pltpu.CompilerParams(needs_layout_passes=False),
        scratch_types=[
            pltpu.VMEM((2, d, _PAD), jnp.float32),
            pltpu.VMEM((2, _PAD, _PAD), jnp.float32),
            pltpu.SemaphoreType.DMA,
            pltpu.SemaphoreType.DMA,
        ],
    )
    def ka(tt_hbm, tail_hbm, pad_hbm, sbuf, tbuf, sem_i, sem_o):
        iota = lax.iota(jnp.int32, _L)
        wid = lax.axis_index("s") * 2 + lax.axis_index("c")
        niter = (ngfull - 1 - wid) // _NW + 1

        def get(i, bb):
            return pltpu.make_async_copy(
                tt_hbm.at[pl.ds(0, d), pl.ds((wid + i * _NW) * _PAD, _PAD)],
                sbuf.at[bb],
                sem_i,
            )

        def put(i, bb):
            return pltpu.make_async_copy(
                tbuf.at[bb],
                pad_hbm.at[pl.ds((wid + i * _NW) * _PAD, _PAD)],
                sem_o,
            )

        get(0, 0).start()

        @pl.loop(0, niter)
        def _(i):
            bb = lax.rem(i, 2)
            get(i, bb).wait()

            @pl.when(i + 1 < niter)
            def _():
                get(i + 1, 1 - bb).start()

            @pl.when(i >= 2)
            def _():
                put(i - 2, bb).wait()

            @pl.loop(0, _PAD)
            def _(r):
                r16 = jnp.full((_L,), r, jnp.int32)
                for db in range(0, _DIM, _L):
                    vals = plsc.load_gather(sbuf.at[bb], [iota + db, r16])
                    tbuf.at[bb, r, pl.ds(db, _L)][...] = vals * _SCALE

            put(i, bb).start()

        @pl.when(niter >= 2)
        def _():
            put(niter - 2, lax.rem(niter, 2)).wait()
        put(niter - 1, lax.rem(niter - 1, 2)).wait()

        if tail:
            @pl.when(wid == 0)
            def _():
                pltpu.async_copy(tail_hbm, sbuf.at[0], sem_i).wait()
                pltpu.async_copy(
                    sbuf.at[0, pl.ds(0, tail)],
                    pad_hbm.at[pl.ds(ngfull * _PAD, tail)],
                    sem_o,
                ).wait()

    return ka(tab_t, tail_pad)


def _gather_rows(tok_flat, tab_pad):
    n = tok_flat.shape[0]
    per_w = n // _NW
    nchunk = per_w // _GCHUNK
    mesh = plsc.VectorSubcoreMesh(core_axis_name="c", subcore_axis_name="s")

    @pl.kernel(
        out_type=jax.ShapeDtypeStruct((n, _DIM), jnp.float32),
        mesh=mesh,
        compiler_params=pltpu.CompilerParams(needs_layout_passes=False),
        scratch_types=[
            pltpu.VMEM((per_w,), jnp.int32),
            pltpu.VMEM((_NBUF, _GCHUNK, _PAD), jnp.float32),
            pltpu.VMEM((_NBUF, _GCHUNK, _DIM), jnp.float32),
            pltpu.SemaphoreType.DMA,
            pltpu.SemaphoreType.DMA,
            pltpu.SemaphoreType.DMA,
        ],
    )
    def kb(tab_hbm, tok_hbm, out_hbm, idx_v, gbuf, obuf, sem_i, sem_g, sem_o):
        wid = lax.axis_index("s") * 2 + lax.axis_index("c")
        base = wid * per_w
        pltpu.async_copy(tok_hbm.at[pl.ds(base, per_w)], idx_v, sem_i).wait()

        def gath(kk, bb):
            return pltpu.make_async_copy(
                tab_hbm.at[idx_v.at[pl.ds(kk * _GCHUNK, _GCHUNK)]],
                gbuf.at[bb],
                sem_g,
            )

        def put(kk, bb):
            return pltpu.make_async_copy(
                obuf.at[bb],
                out_hbm.at[pl.ds(base + kk * _GCHUNK, _GCHUNK)],
                sem_o,
            )

        for bb in range(_NBUF):
            gath(bb, bb).start()

        @pl.loop(0, nchunk, step=_NBUF)
        def _(k0):
            for bb in range(_NBUF):
                kk = k0 + bb
                gath(kk, bb).wait()

                @pl.when(kk >= _NBUF)
                def _():
                    put(kk - _NBUF, bb).wait()

                @pl.loop(0, _GCHUNK)
                def _(rr):
                    for c in range(0, _DIM, _L):
                        obuf.at[bb, rr, pl.ds(c, _L)][...] = gbuf.at[
                            bb, rr, pl.ds(c, _L)
                        ][...]

                put(kk, bb).start()

                @pl.when(kk + _NBUF < nchunk)
                def _():
                    gath(kk + _NBUF, bb).start()

        for bb in range(_NBUF):
            put(nchunk - _NBUF + bb, bb).wait()

    return kb(tab_pad, tok_flat)


def kernel(tokens, table):
    nb, ns = tokens.shape
    v = table.shape[0]
    ntail = v - (v // _PAD) * _PAD
    tail_pad = jnp.pad(
        table[v - ntail :] * _SCALE, ((0, _DIM - ntail), (0, _PAD - _DIM))
    )
    tab_pad = _transpose_scale_pad(table.T, tail_pad)
    tok_flat = tokens.astype(jnp.int32).reshape(nb * ns)
    out = _gather_rows(tok_flat, tab_pad)
    return out.reshape(nb, ns, _DIM)


# speedup vs baseline: 2.2497x; 2.2497x over previous
import jax
import jax.numpy as jnp
from jax import lax
from jax.experimental import pallas as pl
from jax.experimental.pallas import tpu as pltpu
from jax.experimental.pallas import tpu_sc as plsc

_DIM = 64
_CHUNK = 128
_NBUF = 2
_SCALE = 8.0
_L = 16
_NW = 32


def _sc_embed(tok_flat, table):
    n = tok_flat.shape[0]
    per_w = n // _NW
    nchunk = per_w // _CHUNK
    prow = _CHUNK // 2
    mesh = plsc.VectorSubcoreMesh(core_axis_name="c", subcore_axis_name="s")

    @pl.kernel(
        out_type=jax.ShapeDtypeStruct((n // 2, 2 * _DIM), jnp.float32),
        mesh=mesh,
        compiler_params=pltpu.CompilerParams(use_tc_tiling_on_sc=False),
        scratch_types=[
            pltpu.VMEM((per_w,), jnp.int32),
            pltpu.VMEM((_NBUF, _CHUNK, _DIM), jnp.float32),
            pltpu.VMEM((_NBUF, prow, 2 * _DIM), jnp.float32),
            pltpu.SemaphoreType.DMA,
            pltpu.SemaphoreType.DMA,
            pltpu.SemaphoreType.DMA,
        ],
    )
    def k(tab_hbm, tok_hbm, out_hbm, idx_v, gbuf, obuf, sem_i, sem_g, sem_o):
        wid = lax.axis_index("s") * 2 + lax.axis_index("c")
        base = wid * per_w
        pltpu.async_copy(tok_hbm.at[pl.ds(base, per_w)], idx_v, sem_i).wait()

        def gather(kk, b):
            return pltpu.make_async_copy(
                tab_hbm.at[idx_v.at[pl.ds(kk * _CHUNK, _CHUNK)]],
                gbuf.at[b],
                sem_g,
            )

        def put(kk, b):
            return pltpu.make_async_copy(
                obuf.at[b],
                out_hbm.at[pl.ds((base + kk * _CHUNK) // 2, prow)],
                sem_o,
            )

        for b in range(_NBUF):
            gather(b, b).start()

        @pl.loop(0, nchunk, step=_NBUF)
        def _(k0):
            for b in range(_NBUF):
                kk = k0 + b
                gather(kk, b).wait()

                @pl.when(kk >= _NBUF)
                def _():
                    put(kk - _NBUF, b).wait()

                @pl.loop(0, prow)
                def _(r):
                    for p in range(2):
                        for c in range(0, _DIM, _L):
                            obuf.at[b, r, pl.ds(p * _DIM + c, _L)][...] = (
                                gbuf.at[b, 2 * r + p, pl.ds(c, _L)][...]
                                * _SCALE
                            )

                put(kk, b).start()

                @pl.when(kk + _NBUF < nchunk)
                def _():
                    gather(kk + _NBUF, b).start()

        for b in range(_NBUF):
            put(nchunk - _NBUF + b, b).wait()

    return k(table, tok_flat)


def kernel(tokens, table):
    b, s = tokens.shape
    tok_flat = tokens.astype(jnp.int32).reshape(b * s)
    out = _sc_embed(tok_flat, table)
    return out.reshape(b, s, _DIM)
